# Initial kernel scaffold; baseline (speedup 1.0000x reference)
import jax
import jax.numpy as jnp
from jax.experimental import pallas as pl


def kernel(x, edge_index, W_self1, W_neigh1, b1, W_self2, W_neigh2, b2):
    raise NotImplementedError("write your pallas kernel here")


# speedup vs baseline: 4.5589x; 4.5589x over previous
import jax
import jax.numpy as jnp
from jax import lax
from jax.experimental import pallas as pl
from jax.experimental.pallas import tpu as pltpu
from jax.experimental.pallas import tpu_sc as plsc

N = 10000
E = 160000
D = 256
H = D // 2

NC = 2
NS = 16
CHUNK = 80
NROWS = E // CHUNK
GROUP = 8
NGROUPS = NROWS // GROUP
GSLOTS = -(-NGROUPS // NS)
HGROUPS = NGROUPS // NC
HSLOTS = -(-HGROUPS // NS)
N_PAD = 10240
RPS = N_PAD // NS

_VMESH = plsc.VectorSubcoreMesh(core_axis_name="c", subcore_axis_name="s")


def _agg_body(h2, srcr, dstr, zeros_agg, agg_out, src_v, dst_v, rows_v,
              agg_sh):
    c = lax.axis_index("c")
    s = lax.axis_index("s")

    pltpu.sync_copy(zeros_agg, rows_v)
    for k in range(RPS // CHUNK):
        off = s * RPS + k * CHUNK
        pltpu.sync_copy(rows_v, agg_sh.at[pl.ds(off, CHUNK)])
    plsc.subcore_barrier()

    @pl.loop(0, GSLOTS)
    def _(g):
        gi = s + g * NS

        @pl.when(gi < NGROUPS)
        def _():
            row = pl.multiple_of(gi * GROUP, GROUP)
            pltpu.sync_copy(srcr.at[c, pl.ds(row, GROUP)], src_v)
            pltpu.sync_copy(dstr.at[pl.ds(row, GROUP)], dst_v)
            for j in range(GROUP):
                pltpu.sync_copy(h2.at[src_v.at[j]], rows_v)
                pltpu.sync_copy(rows_v, agg_sh.at[dst_v.at[j]], add=True)

    plsc.subcore_barrier()

    for k in range(RPS // CHUNK):
        sl = pl.ds(s * RPS + k * CHUNK, CHUNK)
        pltpu.sync_copy(agg_sh.at[sl], rows_v)
        pltpu.sync_copy(rows_v, agg_out.at[c, sl])


def _deg_body(dstr, ones_hbm, zeros_agg, deg_out, dst_v, buf_v, deg_sh):
    c = lax.axis_index("c")
    s = lax.axis_index("s")

    pltpu.sync_copy(zeros_agg, buf_v)
    for k in range(RPS // CHUNK):
        off = s * RPS + k * CHUNK
        pltpu.sync_copy(buf_v, deg_sh.at[pl.ds(off, CHUNK)])
    pltpu.sync_copy(ones_hbm, buf_v)
    plsc.subcore_barrier()

    @pl.loop(0, HSLOTS)
    def _(g):
        lg = s + g * NS

        @pl.when(lg < HGROUPS)
        def _():
            row = pl.multiple_of((c * HGROUPS + lg) * GROUP, GROUP)
            pltpu.sync_copy(dstr.at[pl.ds(row, GROUP)], dst_v)
            for j in range(GROUP):
                pltpu.sync_copy(buf_v, deg_sh.at[dst_v.at[j]], add=True)

    plsc.subcore_barrier()

    for k in range(RPS // CHUNK):
        sl = pl.ds(s * RPS + k * CHUNK, CHUNK)
        pltpu.sync_copy(deg_sh.at[sl], buf_v)
        pltpu.sync_copy(buf_v, deg_out.at[c, sl])


_AGG_OUT = [jax.ShapeDtypeStruct((NC, N_PAD, H), jnp.float32)]
_AGG_SCRATCH = [
    pltpu.VMEM((GROUP, CHUNK), jnp.int32),
    pltpu.VMEM((GROUP, CHUNK), jnp.int32),
    pltpu.VMEM((CHUNK, H), jnp.float32),
    pltpu.VMEM_SHARED((N_PAD, H), jnp.float32),
]
_DEG_SCRATCH = [
    pltpu.VMEM((GROUP, CHUNK), jnp.int32),
    pltpu.VMEM((CHUNK, H), jnp.float32),
    pltpu.VMEM_SHARED((N_PAD, H), jnp.float32),
]


def _first(out):
    return out[0] if isinstance(out, (tuple, list)) else out


def _sc_agg(h2, srcr2, dstr, zeros_agg):
    kern = pl.kernel(_agg_body, out_type=_AGG_OUT, mesh=_VMESH,
                     scratch_types=_AGG_SCRATCH)
    return _first(kern(h2, srcr2, dstr, zeros_agg))


def _sc_deg(dstr, ones, zeros_agg):
    kern = pl.kernel(_deg_body, out_type=_AGG_OUT, mesh=_VMESH,
                     scratch_types=_DEG_SCRATCH)
    return _first(kern(dstr, ones, zeros_agg))


BLK = 1000


def _inv_deg(deg_ref):
    return 1.0 / jnp.maximum(deg_ref[0][:, 0:1] + deg_ref[1][:, 0:1], 1.0)


def _l1_body(x_ref, agg_ref, deg_ref, ws_ref, wn_ref, b_ref, out_ref):
    inv = _inv_deg(deg_ref)
    m0 = agg_ref[0] * inv
    m1 = agg_ref[1] * inv
    acc = jnp.dot(x_ref[...], ws_ref[...], preferred_element_type=jnp.float32)
    acc += jnp.dot(m0, wn_ref[0:H, :], preferred_element_type=jnp.float32)
    acc += jnp.dot(m1, wn_ref[H:D, :], preferred_element_type=jnp.float32)
    acc += b_ref[...]
    acc = jnp.maximum(acc, 0.0)
    out_ref[0] = acc[:, 0:H]
    out_ref[1] = acc[:, H:D]


def _l2_body(h_ref, agg_ref, deg_ref, ws_ref, wn_ref, b_ref, out_ref):
    inv = _inv_deg(deg_ref)
    m0 = agg_ref[0] * inv
    m1 = agg_ref[1] * inv
    acc = jnp.dot(h_ref[0], ws_ref[0:H, :], preferred_element_type=jnp.float32)
    acc += jnp.dot(h_ref[1], ws_ref[H:D, :], preferred_element_type=jnp.float32)
    acc += jnp.dot(m0, wn_ref[0:H, :], preferred_element_type=jnp.float32)
    acc += jnp.dot(m1, wn_ref[H:D, :], preferred_element_type=jnp.float32)
    acc += b_ref[...]
    out_ref[...] = acc


_W_SPECS = [
    pl.BlockSpec((D, D), lambda i: (0, 0)),
    pl.BlockSpec((D, D), lambda i: (0, 0)),
    pl.BlockSpec((1, D), lambda i: (0, 0)),
]
_HALVES_SPEC = pl.BlockSpec((NC, BLK, H), lambda i: (0, i, 0))


def _tc_layer1(x, agg, deg, ws, wn, b2d):
    return pl.pallas_call(
        _l1_body,
        grid=(N // BLK,),
        in_specs=[pl.BlockSpec((BLK, D), lambda i: (i, 0)),
                  _HALVES_SPEC, _HALVES_SPEC] + _W_SPECS,
        out_specs=_HALVES_SPEC,
        out_shape=jax.ShapeDtypeStruct((NC, N, H), jnp.float32),
    )(x, agg, deg, ws, wn, b2d)


def _tc_layer2(h, agg, deg, ws, wn, b2d):
    return pl.pallas_call(
        _l2_body,
        grid=(N // BLK,),
        in_specs=[_HALVES_SPEC, _HALVES_SPEC, _HALVES_SPEC] + _W_SPECS,
        out_specs=pl.BlockSpec((BLK, D), lambda i: (i, 0)),
        out_shape=jax.ShapeDtypeStruct((N, D), jnp.float32),
    )(h, agg, deg, ws, wn, b2d)


def kernel(x, edge_index, W_self1, W_neigh1, b1, W_self2, W_neigh2, b2):
    src = edge_index[0]
    dst = edge_index[1]
    srcr = src.reshape(NROWS, CHUNK)
    srcr2 = jnp.stack([srcr, srcr + N])
    dstr = dst.reshape(NROWS, CHUNK)
    zeros_agg = jnp.zeros((CHUNK, H), jnp.float32)
    ones = jnp.ones((CHUNK, H), jnp.float32)

    x2 = jnp.stack([x[:, :H], x[:, H:]]).reshape(NC * N, H)
    deg = _sc_deg(dstr, ones, zeros_agg)
    agg1 = _sc_agg(x2, srcr2, dstr, zeros_agg)
    h_halves = _tc_layer1(x, agg1, deg, W_self1, W_neigh1, b1.reshape(1, D))
    h2 = h_halves.reshape(NC * N, H)
    agg2 = _sc_agg(h2, srcr2, dstr, zeros_agg)
    return _tc_layer2(h_halves, agg2, deg, W_self2, W_neigh2, b2.reshape(1, D))
